# baseline (speedup 1.0000x reference)
import functools

import jax
import jax.numpy as jnp
from jax import lax
from jax.experimental import pallas as pl
from jax.experimental.pallas import tpu as pltpu
from jax.experimental.pallas import tpu_sc as plsc

_ROWS = 16384
_COLS = 4096
_OFFSET = 7
_NOUT = 64
_TOTAL = _ROWS * _NOUT
_NC = 2
_NS = 16
_NW = _NC * _NS
_PER_W = _TOTAL // _NW
_NCHUNK = 8
_CHUNK = _PER_W // _NCHUNK

_mesh = plsc.VectorSubcoreMesh(core_axis_name="c", subcore_axis_name="s")


@functools.partial(
    pl.kernel,
    out_type=jax.ShapeDtypeStruct((_TOTAL,), jnp.float32),
    mesh=_mesh,
    scratch_types=[
        pltpu.VMEM((_PER_W,), jnp.int32),
        pltpu.VMEM((_PER_W,), jnp.float32),
        pltpu.SemaphoreType.DMA,
    ],
    compiler_params=pltpu.CompilerParams(
        use_tc_tiling_on_sc=False,
        disable_bounds_checks=True,
        disable_semaphore_checks=True,
    ),
)
def _gather_col(x_hbm, out_hbm, idx_v, vals_v, sem):
    wid = lax.axis_index("s") * _NC + lax.axis_index("c")
    qbase = wid * _PER_W
    lane = lax.iota(jnp.int32, 16)
    pattern = (
        jnp.where(lane >= 8, 32768, 0) + (lane & 7) * 128 + _OFFSET
    )

    copies = []
    for j in range(_NCHUNK):
        cstart = j * _CHUNK

        def body(g, carry, cstart=cstart):
            qb = qbase + cstart + g * 16
            ii = (qb >> 7) & 7
            rt = (qb >> 10) & 127
            it = (qb >> 17) & 7
            s = (
                rt * 524288
                + ((qb >> 3) & 15) * 32768
                + it * 4096
                + (ii >> 1) * 1024
                + (ii & 1) * 64
            )
            idx_v[pl.ds(cstart + g * 16, 16)] = pattern + s
            return carry

        lax.fori_loop(0, _CHUNK // 16, body, 0)
        copies.append(
            pltpu.async_copy(
                x_hbm.at[idx_v.at[pl.ds(cstart, _CHUNK)]],
                vals_v.at[pl.ds(cstart, _CHUNK)],
                sem,
            )
        )
    for c in copies:
        c.wait()
    pltpu.sync_copy(vals_v, out_hbm.at[pl.ds(qbase, _PER_W)])


def kernel(x):
    y = x.reshape(2048, 8, _COLS // 128, 128).transpose(0, 2, 1, 3)
    out = _gather_col(y.reshape(_ROWS * _COLS))
    return (
        out.reshape(_NOUT // 8, _ROWS // 128, 8, 128)
        .transpose(1, 3, 0, 2)
        .reshape(_ROWS, _NOUT)
    )


# speedup vs baseline: 1.0036x; 1.0036x over previous
import functools

import jax
import jax.numpy as jnp
from jax import lax
from jax.experimental import pallas as pl
from jax.experimental.pallas import tpu as pltpu
from jax.experimental.pallas import tpu_sc as plsc

_ROWS = 16384
_COLS = 4096
_OFFSET = 7
_NOUT = 64
_TOTAL = _ROWS * _NOUT
_NC = 2
_NS = 16
_NW = _NC * _NS
_PER_W = _TOTAL // _NW
_NCHUNK = 16
_CHUNK = _PER_W // _NCHUNK

_mesh = plsc.VectorSubcoreMesh(core_axis_name="c", subcore_axis_name="s")


@functools.partial(
    pl.kernel,
    out_type=jax.ShapeDtypeStruct((_TOTAL,), jnp.float32),
    mesh=_mesh,
    scratch_types=[
        pltpu.VMEM((_PER_W,), jnp.int32),
        pltpu.VMEM((_PER_W,), jnp.float32),
        pltpu.SemaphoreType.DMA,
    ],
    compiler_params=pltpu.CompilerParams(
        use_tc_tiling_on_sc=False,
        disable_bounds_checks=True,
        disable_semaphore_checks=True,
        skip_device_barrier=True,
    ),
)
def _gather_col(x_hbm, out_hbm, idx_v, vals_v, sem):
    wid = lax.axis_index("s") * _NC + lax.axis_index("c")
    qbase = wid * _PER_W
    lane = lax.iota(jnp.int32, 16)
    pattern = (
        jnp.where(lane >= 8, 32768, 0) + (lane & 7) * 128 + _OFFSET
    )

    copies = []
    for j in range(_NCHUNK):
        cstart = j * _CHUNK

        def body(g, carry, cstart=cstart):
            qb = qbase + cstart + g * 16
            ii = (qb >> 7) & 7
            rt = (qb >> 10) & 127
            it = (qb >> 17) & 7
            s = (
                rt * 524288
                + ((qb >> 3) & 15) * 32768
                + it * 4096
                + (ii >> 1) * 1024
                + (ii & 1) * 64
            )
            idx_v[pl.ds(cstart + g * 16, 16)] = pattern + s
            return carry

        lax.fori_loop(0, _CHUNK // 16, body, 0)
        copies.append(
            pltpu.async_copy(
                x_hbm.at[idx_v.at[pl.ds(cstart, _CHUNK)]],
                vals_v.at[pl.ds(cstart, _CHUNK)],
                sem,
            )
        )
    for c in copies:
        c.wait()
    pltpu.sync_copy(vals_v, out_hbm.at[pl.ds(qbase, _PER_W)])


def kernel(x):
    y = x.reshape(2048, 8, _COLS // 128, 128).transpose(0, 2, 1, 3)
    out = _gather_col(y.reshape(_ROWS * _COLS))
    return (
        out.reshape(_NOUT // 8, _ROWS // 128, 8, 128)
        .transpose(1, 3, 0, 2)
        .reshape(_ROWS, _NOUT)
    )


# speedup vs baseline: 1.0094x; 1.0059x over previous
import functools

import jax
import jax.numpy as jnp
from jax import lax
from jax.experimental import pallas as pl
from jax.experimental.pallas import tpu as pltpu
from jax.experimental.pallas import tpu_sc as plsc

_ROWS = 16384
_COLS = 4096
_OFFSET = 7
_NOUT = 64
_TOTAL = _ROWS * _NOUT
_NC = 2
_NS = 16
_NW = _NC * _NS
_PER_W = _TOTAL // _NW
_NCHUNK = 16
_CHUNK = _PER_W // _NCHUNK

_mesh = plsc.VectorSubcoreMesh(core_axis_name="c", subcore_axis_name="s")


@functools.partial(
    pl.kernel,
    out_type=jax.ShapeDtypeStruct((_TOTAL,), jnp.float32),
    mesh=_mesh,
    scratch_types=[
        pltpu.VMEM((_PER_W,), jnp.int32),
        pltpu.VMEM((_PER_W,), jnp.float32),
        pltpu.SemaphoreType.DMA((_NCHUNK,)),
        pltpu.SemaphoreType.DMA,
    ],
    compiler_params=pltpu.CompilerParams(
        use_tc_tiling_on_sc=False,
        disable_bounds_checks=True,
        disable_semaphore_checks=True,
        skip_device_barrier=True,
    ),
)
def _gather_col(x_hbm, out_hbm, idx_v, vals_v, sems, wsem):
    wid = lax.axis_index("s") * _NC + lax.axis_index("c")
    qbase = wid * _PER_W
    lane = lax.iota(jnp.int32, 16)
    pattern = (
        jnp.where(lane >= 8, 32768, 0) + (lane & 7) * 128 + _OFFSET
    )

    copies = []
    for j in range(_NCHUNK):
        cstart = j * _CHUNK

        def body(g, carry, cstart=cstart):
            qb = qbase + cstart + g * 16
            ii = (qb >> 7) & 7
            rt = (qb >> 10) & 127
            it = (qb >> 17) & 7
            s = (
                rt * 524288
                + ((qb >> 3) & 15) * 32768
                + it * 4096
                + (ii >> 1) * 1024
                + (ii & 1) * 64
            )
            idx_v[pl.ds(cstart + g * 16, 16)] = pattern + s
            return carry

        lax.fori_loop(0, _CHUNK // 16, body, 0)
        copies.append(
            pltpu.async_copy(
                x_hbm.at[idx_v.at[pl.ds(cstart, _CHUNK)]],
                vals_v.at[pl.ds(cstart, _CHUNK)],
                sems.at[j],
            )
        )
    writes = []
    for j, c in enumerate(copies):
        c.wait()
        writes.append(
            pltpu.async_copy(
                vals_v.at[pl.ds(j * _CHUNK, _CHUNK)],
                out_hbm.at[pl.ds(qbase + j * _CHUNK, _CHUNK)],
                wsem,
            )
        )
    for w in writes:
        w.wait()


def kernel(x):
    y = x.reshape(2048, 8, _COLS // 128, 128).transpose(0, 2, 1, 3)
    out = _gather_col(y.reshape(_ROWS * _COLS))
    return (
        out.reshape(_NOUT // 8, _ROWS // 128, 8, 128)
        .transpose(1, 3, 0, 2)
        .reshape(_ROWS, _NOUT)
    )


# speedup vs baseline: 1.0097x; 1.0002x over previous
import functools

import jax
import jax.numpy as jnp
from jax import lax
from jax.experimental import pallas as pl
from jax.experimental.pallas import tpu as pltpu
from jax.experimental.pallas import tpu_sc as plsc

_ROWS = 16384
_COLS = 4096
_OFFSET = 7
_NOUT = 64
_TOTAL = _ROWS * _NOUT
_NC = 2
_NS = 16
_NW = _NC * _NS
_PER_W = _TOTAL // _NW
_NCHUNK = 16
_CHUNK = _PER_W // _NCHUNK

_mesh = plsc.VectorSubcoreMesh(core_axis_name="c", subcore_axis_name="s")


@functools.partial(
    pl.kernel,
    out_type=jax.ShapeDtypeStruct((_TOTAL,), jnp.float32),
    mesh=_mesh,
    scratch_types=[
        pltpu.VMEM((_PER_W,), jnp.int32),
        pltpu.VMEM((_PER_W,), jnp.float32),
        pltpu.SemaphoreType.DMA((_NCHUNK,)),
        pltpu.SemaphoreType.DMA,
    ],
    compiler_params=pltpu.CompilerParams(use_tc_tiling_on_sc=False),
)
def _gather_col(x_hbm, out_hbm, idx_v, vals_v, sems, wsem):
    wid = lax.axis_index("s") * _NC + lax.axis_index("c")
    qbase = wid * _PER_W
    lane = lax.iota(jnp.int32, 16)
    pattern = (
        jnp.where(lane >= 8, 32768, 0) + (lane & 7) * 128 + _OFFSET
    )

    copies = []
    for j in range(_NCHUNK):
        cstart = j * _CHUNK

        def body(g, carry, cstart=cstart):
            qb = qbase + cstart + g * 16
            ii = (qb >> 7) & 7
            rt = (qb >> 10) & 127
            it = (qb >> 17) & 7
            s = (
                rt * 524288
                + ((qb >> 3) & 15) * 32768
                + it * 4096
                + (ii >> 1) * 1024
                + (ii & 1) * 64
            )
            idx_v[pl.ds(cstart + g * 16, 16)] = pattern + s
            return carry

        lax.fori_loop(0, _CHUNK // 16, body, 0)
        copies.append(
            pltpu.async_copy(
                x_hbm.at[idx_v.at[pl.ds(cstart, _CHUNK)]],
                vals_v.at[pl.ds(cstart, _CHUNK)],
                sems.at[j],
            )
        )
    writes = []
    for j, c in enumerate(copies):
        c.wait()
        writes.append(
            pltpu.async_copy(
                vals_v.at[pl.ds(j * _CHUNK, _CHUNK)],
                out_hbm.at[pl.ds(qbase + j * _CHUNK, _CHUNK)],
                wsem,
            )
        )
    for w in writes:
        w.wait()


def kernel(x):
    y = x.reshape(2048, 8, _COLS // 128, 128).transpose(0, 2, 1, 3)
    out = _gather_col(y.reshape(_ROWS * _COLS))
    return (
        out.reshape(_NOUT // 8, _ROWS // 128, 8, 128)
        .transpose(1, 3, 0, 2)
        .reshape(_ROWS, _NOUT)
    )
